# baseline (speedup 1.0000x reference)
import jax
import jax.numpy as jnp
from jax import lax
from jax.experimental import pallas as pl
from jax.experimental.pallas import tpu as pltpu
from jax.experimental.pallas import tpu_sc as plsc

B, F, V, D, NSEEN = 16384, 26, 1000, 64, 20
FN = F * NSEEN
NP = F // 2
BP = B * NP
NROW = NP * NSEEN * NSEEN
NROWP = 5248
NW = 32
BLK = 128

NCHUNK = 4
CBATCH = B // NCHUNK
CBP = CBATCH * NP
ROWS_W = CBP // NW
NBLK = ROWS_W // BLK
NRING = 4
AHEAD = 2
CB = 512


def _fuse_body(emb_ref, w_ref, def_ref, t_ref):
    w = w_ref[...]
    t_ref[...] = w * emb_ref[...] + (1.0 - w) * def_ref[...]


def _tr_body(p_ref, o_ref):
    x = p_ref[...]
    for p in range(NP):
        o_ref[p, :, :] = x[:, p, :].T


def _tr_body_alias(buf_ref, p_ref, o_ref):
    del buf_ref
    x = p_ref[...]
    for p in range(NP):
        o_ref[p, :, :] = x[:, p, :].T


def _make_gather_body(chunk):
    c0 = chunk * CBP

    def _gather_body(xe_hbm, xo_hbm, tp_hbm, out_hbm,
                     xe0, xe1, xe2, xe3, xo0, xo1, xo2, xo3,
                     i0, i1, i2, i3, r0, r1, r2, r3, tp_sp,
                     gsem0, gsem1, gsem2, gsem3,
                     wsem0, wsem1, wsem2, wsem3):
        wid = lax.axis_index("s") * 2 + lax.axis_index("c")
        sid = lax.axis_index("s")
        lane = lax.broadcasted_iota(jnp.int32, (16,), 0)
        slots = [
            (xe0, xo0, i0, r0, gsem0, wsem0),
            (xe1, xo1, i1, r1, gsem1, wsem1),
            (xe2, xo2, i2, r2, gsem2, wsem2),
            (xe3, xo3, i3, r3, gsem3, wsem3),
        ]

        rows0 = sid * (NROWP // 16)
        pltpu.sync_copy(tp_hbm.at[pl.ds(rows0, NROWP // 16)],
                        tp_sp.at[pl.ds(rows0, NROWP // 16)])
        plsc.subcore_barrier()

        def prep(g):
            xe_b, xo_b, ib, rows_b, gsem, _ = slots[g % NRING]
            base = c0 + wid * ROWS_W + g * BLK
            pltpu.sync_copy(xe_hbm.at[pl.ds(base, BLK)], xe_b)
            pltpu.sync_copy(xo_hbm.at[pl.ds(base, BLK)], xo_b)
            for j in range(BLK // 16):
                xe = xe_b[pl.ds(j * 16, 16)]
                xo = xo_b[pl.ds(j * 16, 16)]
                p = lax.rem(base + j * 16 + lane, NP)
                ib[pl.ds(j * 16, 16)] = (
                    p * (NSEEN * NSEEN) + xe * NSEEN + xo)
            return pltpu.async_copy(tp_sp.at[ib], rows_b, gsem)

        pend_g = {}
        pend_w = {}
        for k in range(min(AHEAD, NBLK)):
            pend_g[k % NRING] = prep(k)
        for g in range(NBLK):
            s = g % NRING
            nxt = g + AHEAD
            if nxt < NBLK:
                s2 = nxt % NRING
                if s2 in pend_w:
                    pend_w[s2].wait()
                    del pend_w[s2]
                pend_g[s2] = prep(nxt)
            pend_g[s].wait()
            del pend_g[s]
            loc = wid * ROWS_W + g * BLK
            pend_w[s] = pltpu.async_copy(
                slots[s][3], out_hbm.at[pl.ds(loc, BLK)], slots[s][5])
        for s in list(pend_w):
            pend_w[s].wait()

    return _gather_body


def kernel(X, emb_w, def_w, w_w):
    emb_e = emb_w[:, :NSEEN, :].reshape(FN, D)
    w_e = w_w[:, :NSEEN, :].reshape(FN, 1)
    def_e = jnp.broadcast_to(def_w[:, None, :], (F, NSEEN, D)).reshape(FN, D)
    t = pl.pallas_call(
        _fuse_body,
        out_shape=jax.ShapeDtypeStruct((FN, D), jnp.float32),
    )(emb_e, w_e, def_e)

    t3 = t.reshape(NP, 2, NSEEN, D)
    te = jnp.broadcast_to(t3[:, 0, :, None, :], (NP, NSEEN, NSEEN, D))
    to = jnp.broadcast_to(t3[:, 1, None, :, :], (NP, NSEEN, NSEEN, D))
    tp = jnp.concatenate([te, to], axis=-1).reshape(NROW, 2 * D)
    tp = jnp.pad(tp, ((0, NROWP - NROW), (0, 0)))

    xe_flat = X[:, 0::2].reshape(BP)
    xo_flat = X[:, 1::2].reshape(BP)

    mesh = plsc.VectorSubcoreMesh(core_axis_name="c", subcore_axis_name="s")
    scratch = (
        [pltpu.VMEM((BLK,), jnp.int32) for _ in range(2 * NRING)]
        + [pltpu.VMEM((BLK,), jnp.int32) for _ in range(NRING)]
        + [pltpu.VMEM((BLK, 2 * D), jnp.float32) for _ in range(NRING)]
        + [pltpu.VMEM_SHARED((NROWP, 2 * D), jnp.float32)]
        + [pltpu.SemaphoreType.DMA for _ in range(2 * NRING)]
    )

    nblk_tr = CBATCH // CB
    buf = None
    for c in range(NCHUNK):
        gather = pl.kernel(
            _make_gather_body(c),
            mesh=mesh,
            out_type=jax.ShapeDtypeStruct((CBP, 2 * D), jnp.float32),
            scratch_types=scratch,
        )
        pc = gather(xe_flat, xo_flat, tp).reshape(CBATCH, NP, 2 * D)
        if buf is None:
            buf = pl.pallas_call(
                _tr_body,
                grid=(nblk_tr,),
                in_specs=[pl.BlockSpec((CB, NP, 2 * D), lambda g: (g, 0, 0))],
                out_specs=pl.BlockSpec((NP, 2 * D, CB), lambda g: (0, 0, g)),
                out_shape=jax.ShapeDtypeStruct((NP, 2 * D, B), jnp.float32),
            )(pc)
        else:
            buf = pl.pallas_call(
                _tr_body_alias,
                grid=(nblk_tr,),
                in_specs=[
                    pl.BlockSpec(memory_space=pltpu.MemorySpace.HBM),
                    pl.BlockSpec((CB, NP, 2 * D), lambda g: (g, 0, 0)),
                ],
                out_specs=pl.BlockSpec(
                    (NP, 2 * D, CB),
                    lambda g, cc=c: (0, 0, cc * nblk_tr + g)),
                out_shape=jax.ShapeDtypeStruct((NP, 2 * D, B), jnp.float32),
                input_output_aliases={0: 0},
            )(buf, pc)

    return jnp.transpose(buf.reshape(F, D, B), (2, 0, 1))


# speedup vs baseline: 1.4661x; 1.4661x over previous
import jax
import jax.numpy as jnp
from jax import lax
from jax.experimental import pallas as pl
from jax.experimental.pallas import tpu as pltpu
from jax.experimental.pallas import tpu_sc as plsc

B, F, V, D, NSEEN = 16384, 26, 1000, 64, 20
FN = F * NSEEN
NP = F // 2
BP = B * NP
NROW = NP * NSEEN * NSEEN
NROWP = 5248
NW = 32
BLK = 128

NCHUNK = 4
CBATCH = B // NCHUNK
CBP = CBATCH * NP
ROWS_W = CBP // NW
NBLK = ROWS_W // BLK
NRING = 4
AHEAD = 2
CB = 512


def _fuse_body(emb_ref, w_ref, def_ref, t_ref):
    w = w_ref[...]
    t_ref[...] = w * emb_ref[...] + (1.0 - w) * def_ref[...]


def _tr_body(p_ref, o_ref):
    x = p_ref[...].reshape(CB, NP, 2 * D)
    for p in range(NP):
        o_ref[p, :, :] = x[:, p, :].T


def _tr_body_alias(buf_ref, p_ref, o_ref):
    del buf_ref
    x = p_ref[...].reshape(CB, NP, 2 * D)
    for p in range(NP):
        o_ref[p, :, :] = x[:, p, :].T


def _make_gather_body(chunk):
    c0 = chunk * CBP

    def _gather_body(xe_hbm, xo_hbm, tp_hbm, out_hbm,
                     xe0, xe1, xe2, xe3, xo0, xo1, xo2, xo3,
                     i0, i1, i2, i3, r0, r1, r2, r3, tp_sp,
                     gsem0, gsem1, gsem2, gsem3,
                     wsem0, wsem1, wsem2, wsem3):
        wid = lax.axis_index("s") * 2 + lax.axis_index("c")
        sid = lax.axis_index("s")
        lane = lax.broadcasted_iota(jnp.int32, (16,), 0)
        slots = [
            (xe0, xo0, i0, r0, gsem0, wsem0),
            (xe1, xo1, i1, r1, gsem1, wsem1),
            (xe2, xo2, i2, r2, gsem2, wsem2),
            (xe3, xo3, i3, r3, gsem3, wsem3),
        ]

        rows0 = sid * (NROWP // 16)
        pltpu.sync_copy(tp_hbm.at[pl.ds(rows0, NROWP // 16)],
                        tp_sp.at[pl.ds(rows0, NROWP // 16)])
        plsc.subcore_barrier()

        def prep(g):
            xe_b, xo_b, ib, rows_b, gsem, _ = slots[g % NRING]
            base = c0 + wid * ROWS_W + g * BLK
            pltpu.sync_copy(xe_hbm.at[pl.ds(base, BLK)], xe_b)
            pltpu.sync_copy(xo_hbm.at[pl.ds(base, BLK)], xo_b)
            for j in range(BLK // 16):
                xe = xe_b[pl.ds(j * 16, 16)]
                xo = xo_b[pl.ds(j * 16, 16)]
                p = lax.rem(base + j * 16 + lane, NP)
                ib[pl.ds(j * 16, 16)] = (
                    p * (NSEEN * NSEEN) + xe * NSEEN + xo)
            return pltpu.async_copy(tp_sp.at[ib], rows_b, gsem)

        pend_g = {}
        pend_w = {}
        for k in range(min(AHEAD, NBLK)):
            pend_g[k % NRING] = prep(k)
        for g in range(NBLK):
            s = g % NRING
            nxt = g + AHEAD
            if nxt < NBLK:
                s2 = nxt % NRING
                if s2 in pend_w:
                    pend_w[s2].wait()
                    del pend_w[s2]
                pend_g[s2] = prep(nxt)
            pend_g[s].wait()
            del pend_g[s]
            loc = wid * ROWS_W + g * BLK
            pend_w[s] = pltpu.async_copy(
                slots[s][3], out_hbm.at[pl.ds(loc, BLK)], slots[s][5])
        for s in list(pend_w):
            pend_w[s].wait()

    return _gather_body


def kernel(X, emb_w, def_w, w_w):
    emb_e = emb_w[:, :NSEEN, :].reshape(FN, D)
    w_e = w_w[:, :NSEEN, :].reshape(FN, 1)
    def_e = jnp.broadcast_to(def_w[:, None, :], (F, NSEEN, D)).reshape(FN, D)
    t = pl.pallas_call(
        _fuse_body,
        out_shape=jax.ShapeDtypeStruct((FN, D), jnp.float32),
    )(emb_e, w_e, def_e)

    t3 = t.reshape(NP, 2, NSEEN, D)
    te = jnp.broadcast_to(t3[:, 0, :, None, :], (NP, NSEEN, NSEEN, D))
    to = jnp.broadcast_to(t3[:, 1, None, :, :], (NP, NSEEN, NSEEN, D))
    tp = jnp.concatenate([te, to], axis=-1).reshape(NROW, 2 * D)
    tp = jnp.pad(tp, ((0, NROWP - NROW), (0, 0)))

    xe_flat = X[:, 0::2].reshape(BP)
    xo_flat = X[:, 1::2].reshape(BP)

    mesh = plsc.VectorSubcoreMesh(core_axis_name="c", subcore_axis_name="s")
    scratch = (
        [pltpu.VMEM((BLK,), jnp.int32) for _ in range(2 * NRING)]
        + [pltpu.VMEM((BLK,), jnp.int32) for _ in range(NRING)]
        + [pltpu.VMEM((BLK, 2 * D), jnp.float32) for _ in range(NRING)]
        + [pltpu.VMEM_SHARED((NROWP, 2 * D), jnp.float32)]
        + [pltpu.SemaphoreType.DMA for _ in range(2 * NRING)]
    )

    nblk_tr = CBATCH // CB
    buf = None
    for c in range(NCHUNK):
        gather = pl.kernel(
            _make_gather_body(c),
            mesh=mesh,
            out_type=jax.ShapeDtypeStruct((CBP, 2 * D), jnp.float32),
            scratch_types=scratch,
        )
        pc = gather(xe_flat, xo_flat, tp)
        if buf is None:
            buf = pl.pallas_call(
                _tr_body,
                grid=(nblk_tr,),
                in_specs=[pl.BlockSpec((CB * NP, 2 * D), lambda g: (g, 0))],
                out_specs=pl.BlockSpec((NP, 2 * D, CB), lambda g: (0, 0, g)),
                out_shape=jax.ShapeDtypeStruct((NP, 2 * D, B), jnp.float32),
            )(pc)
        else:
            buf = pl.pallas_call(
                _tr_body_alias,
                grid=(nblk_tr,),
                in_specs=[
                    pl.BlockSpec(memory_space=pltpu.MemorySpace.HBM),
                    pl.BlockSpec((CB * NP, 2 * D), lambda g: (g, 0)),
                ],
                out_specs=pl.BlockSpec(
                    (NP, 2 * D, CB),
                    lambda g, cc=c: (0, 0, cc * nblk_tr + g)),
                out_shape=jax.ShapeDtypeStruct((NP, 2 * D, B), jnp.float32),
                input_output_aliases={0: 0},
            )(buf, pc)

    return jnp.transpose(buf.reshape(F, D, B), (2, 0, 1))


# speedup vs baseline: 1.7744x; 1.2103x over previous
import jax
import jax.numpy as jnp
from jax import lax
from jax.experimental import pallas as pl
from jax.experimental.pallas import tpu as pltpu
from jax.experimental.pallas import tpu_sc as plsc

B, F, V, D, NSEEN = 16384, 26, 1000, 64, 20
FN = F * NSEEN
NP = F // 2
BP = B * NP
NROW = NP * NSEEN * NSEEN
NROWP = 5248
NW = 32
BLK = 128

NCHUNK = 4
CBATCH = B // NCHUNK
CBP = CBATCH * NP
ROWS_W = CBP // NW
NBLK = ROWS_W // BLK
NRING = 4
AHEAD = 2
CB = 512


def _fuse_body(emb_ref, w_ref, def_ref, t_ref):
    w = w_ref[...]
    t_ref[...] = w * emb_ref[...] + (1.0 - w) * def_ref[...]


def _tr_body(p_ref, o_ref):
    x = p_ref[...].reshape(CB, NP, 2 * D)
    for p in range(NP):
        o_ref[p, :, :] = x[:, p, :].T


def _tr_body_alias(buf_ref, p_ref, o_ref):
    del buf_ref
    x = p_ref[...].reshape(CB, NP, 2 * D)
    for p in range(NP):
        o_ref[p, :, :] = x[:, p, :].T


def _make_gather_body(chunk):
    c0 = chunk * CBP

    def _gather_body(x_hbm, tp_hbm, out_hbm,
                     xc0, xc1, xc2, xc3,
                     i0, i1, i2, i3, r0, r1, r2, r3, tp_sp,
                     gsem0, gsem1, gsem2, gsem3,
                     wsem0, wsem1, wsem2, wsem3):
        wid = lax.axis_index("s") * 2 + lax.axis_index("c")
        sid = lax.axis_index("s")
        lane = lax.broadcasted_iota(jnp.int32, (16,), 0)
        slots = [
            (xc0, i0, r0, gsem0, wsem0),
            (xc1, i1, r1, gsem1, wsem1),
            (xc2, i2, r2, gsem2, wsem2),
            (xc3, i3, r3, gsem3, wsem3),
        ]

        rows0 = sid * (NROWP // 16)
        pltpu.sync_copy(tp_hbm.at[pl.ds(rows0, NROWP // 16)],
                        tp_sp.at[pl.ds(rows0, NROWP // 16)])
        plsc.subcore_barrier()

        def prep(g):
            xc_b, ib, rows_b, gsem, _ = slots[g % NRING]
            base = c0 + wid * ROWS_W + g * BLK
            pltpu.sync_copy(x_hbm.at[pl.ds(2 * base, 2 * BLK)], xc_b)
            for j in range(BLK // 16):
                idx2 = 2 * (j * 16 + lane)
                xe = plsc.load_gather(xc_b, [idx2])
                xo = plsc.load_gather(xc_b, [idx2 + 1])
                p = lax.rem(base + j * 16 + lane, NP)
                ib[pl.ds(j * 16, 16)] = (
                    p * (NSEEN * NSEEN) + xe * NSEEN + xo)
            return pltpu.async_copy(tp_sp.at[ib], rows_b, gsem)

        pend_g = {}
        pend_w = {}
        for k in range(min(AHEAD, NBLK)):
            pend_g[k % NRING] = prep(k)
        for g in range(NBLK):
            s = g % NRING
            nxt = g + AHEAD
            if nxt < NBLK:
                s2 = nxt % NRING
                if s2 in pend_w:
                    pend_w[s2].wait()
                    del pend_w[s2]
                pend_g[s2] = prep(nxt)
            pend_g[s].wait()
            del pend_g[s]
            loc = wid * ROWS_W + g * BLK
            pend_w[s] = pltpu.async_copy(
                slots[s][2], out_hbm.at[pl.ds(loc, BLK)], slots[s][4])
        for s in list(pend_w):
            pend_w[s].wait()

    return _gather_body


def kernel(X, emb_w, def_w, w_w):
    emb_e = emb_w[:, :NSEEN, :].reshape(FN, D)
    w_e = w_w[:, :NSEEN, :].reshape(FN, 1)
    def_e = jnp.broadcast_to(def_w[:, None, :], (F, NSEEN, D)).reshape(FN, D)
    t = pl.pallas_call(
        _fuse_body,
        out_shape=jax.ShapeDtypeStruct((FN, D), jnp.float32),
    )(emb_e, w_e, def_e)

    t3 = t.reshape(NP, 2, NSEEN, D)
    te = jnp.broadcast_to(t3[:, 0, :, None, :], (NP, NSEEN, NSEEN, D))
    to = jnp.broadcast_to(t3[:, 1, None, :, :], (NP, NSEEN, NSEEN, D))
    tp = jnp.concatenate([te, to], axis=-1).reshape(NROW, 2 * D)
    tp = jnp.pad(tp, ((0, NROWP - NROW), (0, 0)))

    x_flat = X.reshape(B * F)

    mesh = plsc.VectorSubcoreMesh(core_axis_name="c", subcore_axis_name="s")
    scratch = (
        [pltpu.VMEM((2 * BLK,), jnp.int32) for _ in range(NRING)]
        + [pltpu.VMEM((BLK,), jnp.int32) for _ in range(NRING)]
        + [pltpu.VMEM((BLK, 2 * D), jnp.float32) for _ in range(NRING)]
        + [pltpu.VMEM_SHARED((NROWP, 2 * D), jnp.float32)]
        + [pltpu.SemaphoreType.DMA for _ in range(2 * NRING)]
    )

    nblk_tr = CBATCH // CB
    buf = None
    for c in range(NCHUNK):
        gather = pl.kernel(
            _make_gather_body(c),
            mesh=mesh,
            out_type=jax.ShapeDtypeStruct((CBP, 2 * D), jnp.float32),
            scratch_types=scratch,
            compiler_params=pltpu.CompilerParams(needs_layout_passes=False),
        )
        pc = gather(x_flat, tp)
        if buf is None:
            buf = pl.pallas_call(
                _tr_body,
                grid=(nblk_tr,),
                in_specs=[pl.BlockSpec((CB * NP, 2 * D), lambda g: (g, 0))],
                out_specs=pl.BlockSpec((NP, 2 * D, CB), lambda g: (0, 0, g)),
                out_shape=jax.ShapeDtypeStruct((NP, 2 * D, B), jnp.float32),
            )(pc)
        else:
            buf = pl.pallas_call(
                _tr_body_alias,
                grid=(nblk_tr,),
                in_specs=[
                    pl.BlockSpec(memory_space=pltpu.MemorySpace.HBM),
                    pl.BlockSpec((CB * NP, 2 * D), lambda g: (g, 0)),
                ],
                out_specs=pl.BlockSpec(
                    (NP, 2 * D, CB),
                    lambda g, cc=c: (0, 0, cc * nblk_tr + g)),
                out_shape=jax.ShapeDtypeStruct((NP, 2 * D, B), jnp.float32),
                input_output_aliases={0: 0},
            )(buf, pc)

    return jnp.transpose(buf.reshape(F, D, B), (2, 0, 1))


# speedup vs baseline: 1.8222x; 1.0269x over previous
import jax
import jax.numpy as jnp
from jax import lax
from jax.experimental import pallas as pl
from jax.experimental.pallas import tpu as pltpu
from jax.experimental.pallas import tpu_sc as plsc

B, F, V, D, NSEEN = 16384, 26, 1000, 64, 20
FN = F * NSEEN
NP = F // 2
BP = B * NP
NROW = NP * NSEEN * NSEEN
NROWP = 5248
NW = 32
BLK = 128
XSEG = BLK * F

NCHUNK = 4
CBATCH = B // NCHUNK
CBP = CBATCH * NP
ROWS_W = CBP // NW
NBLK = ROWS_W // BLK
NRING = 4
AHEAD = 2
CB = 512


def _fuse_body(emb_ref, w_ref, def_ref, t_ref):
    w = w_ref[...]
    t_ref[...] = w * emb_ref[...] + (1.0 - w) * def_ref[...]


def _tr_body(p_ref, o_ref):
    x = p_ref[...]
    for p in range(NP):
        o_ref[p, :, :] = x[p, :, :].T


def _tr_body_alias(buf_ref, p_ref, o_ref):
    del buf_ref
    x = p_ref[...]
    for p in range(NP):
        o_ref[p, :, :] = x[p, :, :].T


def _make_gather_body(chunk):
    def _gather_body(x_hbm, tp_hbm, out_hbm,
                     xc0, xc1, xc2, xc3,
                     i0, i1, i2, i3, r0, r1, r2, r3, tp_sp,
                     xsem0, xsem1, xsem2, xsem3,
                     gsem0, gsem1, gsem2, gsem3,
                     wsem0, wsem1, wsem2, wsem3):
        wid = lax.axis_index("s") * 2 + lax.axis_index("c")
        sid = lax.axis_index("s")
        lane = lax.broadcasted_iota(jnp.int32, (16,), 0)
        slots = [
            (xc0, i0, r0, xsem0, gsem0, wsem0),
            (xc1, i1, r1, xsem1, gsem1, wsem1),
            (xc2, i2, r2, xsem2, gsem2, wsem2),
            (xc3, i3, r3, xsem3, gsem3, wsem3),
        ]

        trow0 = sid * (NROWP // 16)
        pltpu.sync_copy(tp_hbm.at[pl.ds(trow0, NROWP // 16)],
                        tp_sp.at[pl.ds(trow0, NROWP // 16)])
        plsc.subcore_barrier()

        def fire_x(g):
            xc_b, _, _, xsem, _, _ = slots[g % NRING]
            row0 = wid * ROWS_W + g * BLK
            b_abs = chunk * CBATCH + lax.rem(row0, CBATCH)
            return pltpu.async_copy(
                x_hbm.at[pl.ds(b_abs * F, XSEG)], xc_b, xsem)

        def fire_gather(g):
            xc_b, ib, rows_b, _, gsem, _ = slots[g % NRING]
            row0 = wid * ROWS_W + g * BLK
            p = lax.div(row0, CBATCH)
            for j in range(BLK // 16):
                k = j * 16 + lane
                pos = k * F + 2 * p
                xe = plsc.load_gather(xc_b, [pos])
                xo = plsc.load_gather(xc_b, [pos + 1])
                ib[pl.ds(j * 16, 16)] = (
                    p * (NSEEN * NSEEN) + xe * NSEEN + xo)
            return pltpu.async_copy(tp_sp.at[ib], rows_b, gsem)

        pend_x = {}
        pend_g = {}
        pend_w = {}
        for k in range(min(AHEAD, NBLK)):
            pend_x[k % NRING] = fire_x(k)
        pend_x[0].wait()
        del pend_x[0]
        pend_g[0] = fire_gather(0)
        for g in range(NBLK):
            s = g % NRING
            nxt_x = g + AHEAD
            if nxt_x < NBLK:
                pend_x[nxt_x % NRING] = fire_x(nxt_x)
            nxt = g + 1
            if nxt < NBLK:
                s2 = nxt % NRING
                if s2 in pend_w:
                    pend_w[s2].wait()
                    del pend_w[s2]
                pend_x[s2].wait()
                del pend_x[s2]
                pend_g[s2] = fire_gather(nxt)
            pend_g[s].wait()
            del pend_g[s]
            row0 = wid * ROWS_W + g * BLK
            pend_w[s] = pltpu.async_copy(
                slots[s][2], out_hbm.at[pl.ds(row0, BLK)], slots[s][5])
        for s in list(pend_w):
            pend_w[s].wait()

    return _gather_body


def kernel(X, emb_w, def_w, w_w):
    emb_e = emb_w[:, :NSEEN, :].reshape(FN, D)
    w_e = w_w[:, :NSEEN, :].reshape(FN, 1)
    def_e = jnp.broadcast_to(def_w[:, None, :], (F, NSEEN, D)).reshape(FN, D)
    t = pl.pallas_call(
        _fuse_body,
        out_shape=jax.ShapeDtypeStruct((FN, D), jnp.float32),
    )(emb_e, w_e, def_e)

    t3 = t.reshape(NP, 2, NSEEN, D)
    te = jnp.broadcast_to(t3[:, 0, :, None, :], (NP, NSEEN, NSEEN, D))
    to = jnp.broadcast_to(t3[:, 1, None, :, :], (NP, NSEEN, NSEEN, D))
    tp = jnp.concatenate([te, to], axis=-1).reshape(NROW, 2 * D)
    tp = jnp.pad(tp, ((0, NROWP - NROW), (0, 0)))

    x_flat = X.reshape(B * F)

    mesh = plsc.VectorSubcoreMesh(core_axis_name="c", subcore_axis_name="s")
    scratch = (
        [pltpu.VMEM((XSEG,), jnp.int32) for _ in range(NRING)]
        + [pltpu.VMEM((BLK,), jnp.int32) for _ in range(NRING)]
        + [pltpu.VMEM((BLK, 2 * D), jnp.float32) for _ in range(NRING)]
        + [pltpu.VMEM_SHARED((NROWP, 2 * D), jnp.float32)]
        + [pltpu.SemaphoreType.DMA for _ in range(3 * NRING)]
    )

    nblk_tr = CBATCH // CB
    buf = None
    for c in range(NCHUNK):
        gather = pl.kernel(
            _make_gather_body(c),
            mesh=mesh,
            out_type=jax.ShapeDtypeStruct((CBP, 2 * D), jnp.float32),
            scratch_types=scratch,
            compiler_params=pltpu.CompilerParams(needs_layout_passes=False),
        )
        pc = gather(x_flat, tp).reshape(NP, CBATCH, 2 * D)
        if buf is None:
            buf = pl.pallas_call(
                _tr_body,
                grid=(nblk_tr,),
                in_specs=[pl.BlockSpec((NP, CB, 2 * D), lambda g: (0, g, 0))],
                out_specs=pl.BlockSpec((NP, 2 * D, CB), lambda g: (0, 0, g)),
                out_shape=jax.ShapeDtypeStruct((NP, 2 * D, B), jnp.float32),
            )(pc)
        else:
            buf = pl.pallas_call(
                _tr_body_alias,
                grid=(nblk_tr,),
                in_specs=[
                    pl.BlockSpec(memory_space=pltpu.MemorySpace.HBM),
                    pl.BlockSpec((NP, CB, 2 * D), lambda g: (0, g, 0)),
                ],
                out_specs=pl.BlockSpec(
                    (NP, 2 * D, CB),
                    lambda g, cc=c: (0, 0, cc * nblk_tr + g)),
                out_shape=jax.ShapeDtypeStruct((NP, 2 * D, B), jnp.float32),
                input_output_aliases={0: 0},
            )(buf, pc)

    return jnp.transpose(buf.reshape(F, D, B), (2, 0, 1))


# speedup vs baseline: 1.9370x; 1.0630x over previous
import jax
import jax.numpy as jnp
import numpy as np
from jax import lax
from jax.experimental import pallas as pl
from jax.experimental.pallas import tpu as pltpu
from jax.experimental.pallas import tpu_sc as plsc

B, F, V, D, NSEEN = 16384, 26, 1000, 64, 20
FN = F * NSEEN
NP = F // 2
BP = B * NP
NROW = NP * NSEEN * NSEEN
NROWP = 5248
NW = 32
BLK = 128
XSEG = BLK * F

NCHUNK = 4
CBATCH = B // NCHUNK
CBP = CBATCH * NP
ROWS_W = CBP // NW
NBLK = ROWS_W // BLK
NRING = 4
AHEAD = 2
CB = 512


def _fuse_body(emb_ref, w_ref, def_ref, t_ref):
    w = w_ref[...]
    t_ref[...] = w * emb_ref[...] + (1.0 - w) * def_ref[...]


XB = 2048


def _idx_body(x_ref, m_ref, o_ref):
    x = x_ref[...].astype(jnp.float32)
    v = jnp.dot(x, m_ref[...], preferred_element_type=jnp.float32)
    offs = (lax.broadcasted_iota(jnp.int32, (XB, NP), 1)
            * (NSEEN * NSEEN)).astype(jnp.float32)
    o_ref[...] = (v + offs).astype(jnp.int32).T


def _tr_body(p_ref, o_ref):
    x = p_ref[...]
    for p in range(NP):
        o_ref[p, :, :] = x[p, :, :].T


def _tr_body_alias(buf_ref, p_ref, o_ref):
    del buf_ref
    x = p_ref[...]
    for p in range(NP):
        o_ref[p, :, :] = x[p, :, :].T


def _make_gather_body(chunk):
    def _gather_body(x_hbm, tp_hbm, out_hbm,
                     i0, i1, i2, i3, r0, r1, r2, r3, tp_sp,
                     xsem0, xsem1, xsem2, xsem3,
                     gsem0, gsem1, gsem2, gsem3,
                     wsem0, wsem1, wsem2, wsem3):
        wid = lax.axis_index("s") * 2 + lax.axis_index("c")
        sid = lax.axis_index("s")
        slots = [
            (None, i0, r0, xsem0, gsem0, wsem0),
            (None, i1, r1, xsem1, gsem1, wsem1),
            (None, i2, r2, xsem2, gsem2, wsem2),
            (None, i3, r3, xsem3, gsem3, wsem3),
        ]

        trow0 = sid * (NROWP // 16)
        pltpu.sync_copy(tp_hbm.at[pl.ds(trow0, NROWP // 16)],
                        tp_sp.at[pl.ds(trow0, NROWP // 16)])
        plsc.subcore_barrier()

        def fire_x(g):
            _, ib, _, xsem, _, _ = slots[g % NRING]
            row0 = wid * ROWS_W + g * BLK
            p = lax.div(row0, CBATCH)
            b0 = lax.rem(row0, CBATCH)
            pos = p * B + chunk * CBATCH + b0
            return pltpu.async_copy(x_hbm.at[pl.ds(pos, BLK)], ib, xsem)

        def fire_gather(g):
            _, ib, rows_b, _, gsem, _ = slots[g % NRING]
            return pltpu.async_copy(tp_sp.at[ib], rows_b, gsem)

        pend_x = {}
        pend_g = {}
        pend_w = {}
        for k in range(min(AHEAD, NBLK)):
            pend_x[k % NRING] = fire_x(k)
        pend_x[0].wait()
        del pend_x[0]
        pend_g[0] = fire_gather(0)
        for g in range(NBLK):
            s = g % NRING
            nxt_x = g + AHEAD
            if nxt_x < NBLK:
                pend_x[nxt_x % NRING] = fire_x(nxt_x)
            nxt = g + 1
            if nxt < NBLK:
                s2 = nxt % NRING
                if s2 in pend_w:
                    pend_w[s2].wait()
                    del pend_w[s2]
                pend_x[s2].wait()
                del pend_x[s2]
                pend_g[s2] = fire_gather(nxt)
            pend_g[s].wait()
            del pend_g[s]
            row0 = wid * ROWS_W + g * BLK
            pend_w[s] = pltpu.async_copy(
                slots[s][2], out_hbm.at[pl.ds(row0, BLK)], slots[s][5])
        for s in list(pend_w):
            pend_w[s].wait()

    return _gather_body


def kernel(X, emb_w, def_w, w_w):
    emb_e = emb_w[:, :NSEEN, :].reshape(FN, D)
    w_e = w_w[:, :NSEEN, :].reshape(FN, 1)
    def_e = jnp.broadcast_to(def_w[:, None, :], (F, NSEEN, D)).reshape(FN, D)
    t = pl.pallas_call(
        _fuse_body,
        out_shape=jax.ShapeDtypeStruct((FN, D), jnp.float32),
    )(emb_e, w_e, def_e)

    t3 = t.reshape(NP, 2, NSEEN, D)
    te = jnp.broadcast_to(t3[:, 0, :, None, :], (NP, NSEEN, NSEEN, D))
    to = jnp.broadcast_to(t3[:, 1, None, :, :], (NP, NSEEN, NSEEN, D))
    tp = jnp.concatenate([te, to], axis=-1).reshape(NROW, 2 * D)
    tp = jnp.pad(tp, ((0, NROWP - NROW), (0, 0)))

    m = np.zeros((F, NP), dtype=np.float32)
    for p_ in range(NP):
        m[2 * p_, p_] = float(NSEEN)
        m[2 * p_ + 1, p_] = 1.0
    idx_t = pl.pallas_call(
        _idx_body,
        grid=(B // XB,),
        in_specs=[
            pl.BlockSpec((XB, F), lambda g: (g, 0)),
            pl.BlockSpec((F, NP), lambda g: (0, 0)),
        ],
        out_specs=pl.BlockSpec((NP, XB), lambda g: (0, g)),
        out_shape=jax.ShapeDtypeStruct((NP, B), jnp.int32),
    )(X, jnp.asarray(m))
    idx_flat = idx_t.reshape(NP * B)

    mesh = plsc.VectorSubcoreMesh(core_axis_name="c", subcore_axis_name="s")
    scratch = (
        [pltpu.VMEM((BLK,), jnp.int32) for _ in range(NRING)]
        + [pltpu.VMEM((BLK, 2 * D), jnp.float32) for _ in range(NRING)]
        + [pltpu.VMEM_SHARED((NROWP, 2 * D), jnp.float32)]
        + [pltpu.SemaphoreType.DMA for _ in range(3 * NRING)]
    )

    nblk_tr = CBATCH // CB
    buf = None
    for c in range(NCHUNK):
        gather = pl.kernel(
            _make_gather_body(c),
            mesh=mesh,
            out_type=jax.ShapeDtypeStruct((CBP, 2 * D), jnp.float32),
            scratch_types=scratch,
            compiler_params=pltpu.CompilerParams(needs_layout_passes=False),
        )
        pc = gather(idx_flat, tp).reshape(NP, CBATCH, 2 * D)
        if buf is None:
            buf = pl.pallas_call(
                _tr_body,
                grid=(nblk_tr,),
                in_specs=[pl.BlockSpec((NP, CB, 2 * D), lambda g: (0, g, 0))],
                out_specs=pl.BlockSpec((NP, 2 * D, CB), lambda g: (0, 0, g)),
                out_shape=jax.ShapeDtypeStruct((NP, 2 * D, B), jnp.float32),
            )(pc)
        else:
            buf = pl.pallas_call(
                _tr_body_alias,
                grid=(nblk_tr,),
                in_specs=[
                    pl.BlockSpec(memory_space=pltpu.MemorySpace.HBM),
                    pl.BlockSpec((NP, CB, 2 * D), lambda g: (0, g, 0)),
                ],
                out_specs=pl.BlockSpec(
                    (NP, 2 * D, CB),
                    lambda g, cc=c: (0, 0, cc * nblk_tr + g)),
                out_shape=jax.ShapeDtypeStruct((NP, 2 * D, B), jnp.float32),
                input_output_aliases={0: 0},
            )(buf, pc)

    return jnp.transpose(buf.reshape(F, D, B), (2, 0, 1))


# speedup vs baseline: 1.9580x; 1.0109x over previous
import jax
import jax.numpy as jnp
import numpy as np
from jax import lax
from jax.experimental import pallas as pl
from jax.experimental.pallas import tpu as pltpu
from jax.experimental.pallas import tpu_sc as plsc

B, F, V, D, NSEEN = 16384, 26, 1000, 64, 20
FN = F * NSEEN
NP = F // 2
BP = B * NP
NROW = NP * NSEEN * NSEEN
NROWP = 5248
NW = 32
BLK = 128
XSEG = BLK * F

NCHUNK = 4
CBATCH = B // NCHUNK
CBP = CBATCH * NP
ROWS_W = CBP // NW
NBLK = ROWS_W // BLK
NRING = 4
AHEAD = 2
CB = 1024


def _fuse_body(emb_ref, w_ref, def_ref, t_ref):
    w = w_ref[...]
    t_ref[...] = w * emb_ref[...] + (1.0 - w) * def_ref[...]


XB = 2048


def _idx_body(x_ref, m_ref, o_ref):
    x = x_ref[...].astype(jnp.float32)
    v = jnp.dot(x, m_ref[...], preferred_element_type=jnp.float32)
    offs = (lax.broadcasted_iota(jnp.int32, (XB, NP), 1)
            * (NSEEN * NSEEN)).astype(jnp.float32)
    o_ref[...] = (v + offs).astype(jnp.int32).T


def _tr_body(p_ref, o_ref):
    x = p_ref[...]
    for p in range(NP):
        o_ref[p, :, :] = x[p, :, :].T


def _tr_body_alias(buf_ref, p_ref, o_ref):
    del buf_ref
    x = p_ref[...]
    for p in range(NP):
        o_ref[p, :, :] = x[p, :, :].T


def _make_gather_body(chunk):
    def _gather_body(x_hbm, tp_hbm, out_hbm,
                     i0, i1, i2, i3, r0, r1, r2, r3, tp_sp,
                     xsem0, xsem1, xsem2, xsem3,
                     gsem0, gsem1, gsem2, gsem3,
                     wsem0, wsem1, wsem2, wsem3):
        wid = lax.axis_index("s") * 2 + lax.axis_index("c")
        sid = lax.axis_index("s")
        slots = [
            (None, i0, r0, xsem0, gsem0, wsem0),
            (None, i1, r1, xsem1, gsem1, wsem1),
            (None, i2, r2, xsem2, gsem2, wsem2),
            (None, i3, r3, xsem3, gsem3, wsem3),
        ]

        trow0 = sid * (NROWP // 16)
        pltpu.sync_copy(tp_hbm.at[pl.ds(trow0, NROWP // 16)],
                        tp_sp.at[pl.ds(trow0, NROWP // 16)])
        plsc.subcore_barrier()

        def fire_x(g):
            _, ib, _, xsem, _, _ = slots[g % NRING]
            row0 = wid * ROWS_W + g * BLK
            p = lax.div(row0, CBATCH)
            b0 = lax.rem(row0, CBATCH)
            pos = p * B + chunk * CBATCH + b0
            return pltpu.async_copy(x_hbm.at[pl.ds(pos, BLK)], ib, xsem)

        def fire_gather(g):
            _, ib, rows_b, _, gsem, _ = slots[g % NRING]
            return pltpu.async_copy(tp_sp.at[ib], rows_b, gsem)

        pend_x = {}
        pend_g = {}
        pend_w = {}
        for k in range(min(AHEAD, NBLK)):
            pend_x[k % NRING] = fire_x(k)
        pend_x[0].wait()
        del pend_x[0]
        pend_g[0] = fire_gather(0)
        for g in range(NBLK):
            s = g % NRING
            nxt_x = g + AHEAD
            if nxt_x < NBLK:
                pend_x[nxt_x % NRING] = fire_x(nxt_x)
            nxt = g + 1
            if nxt < NBLK:
                s2 = nxt % NRING
                if s2 in pend_w:
                    pend_w[s2].wait()
                    del pend_w[s2]
                pend_x[s2].wait()
                del pend_x[s2]
                pend_g[s2] = fire_gather(nxt)
            pend_g[s].wait()
            del pend_g[s]
            row0 = wid * ROWS_W + g * BLK
            pend_w[s] = pltpu.async_copy(
                slots[s][2], out_hbm.at[pl.ds(row0, BLK)], slots[s][5])
        for s in list(pend_w):
            pend_w[s].wait()

    return _gather_body


def kernel(X, emb_w, def_w, w_w):
    emb_e = emb_w[:, :NSEEN, :].reshape(FN, D)
    w_e = w_w[:, :NSEEN, :].reshape(FN, 1)
    def_e = jnp.broadcast_to(def_w[:, None, :], (F, NSEEN, D)).reshape(FN, D)
    t = pl.pallas_call(
        _fuse_body,
        out_shape=jax.ShapeDtypeStruct((FN, D), jnp.float32),
    )(emb_e, w_e, def_e)

    t3 = t.reshape(NP, 2, NSEEN, D)
    te = jnp.broadcast_to(t3[:, 0, :, None, :], (NP, NSEEN, NSEEN, D))
    to = jnp.broadcast_to(t3[:, 1, None, :, :], (NP, NSEEN, NSEEN, D))
    tp = jnp.concatenate([te, to], axis=-1).reshape(NROW, 2 * D)
    tp = jnp.pad(tp, ((0, NROWP - NROW), (0, 0)))

    m = np.zeros((F, NP), dtype=np.float32)
    for p_ in range(NP):
        m[2 * p_, p_] = float(NSEEN)
        m[2 * p_ + 1, p_] = 1.0
    idx_t = pl.pallas_call(
        _idx_body,
        grid=(B // XB,),
        in_specs=[
            pl.BlockSpec((XB, F), lambda g: (g, 0)),
            pl.BlockSpec((F, NP), lambda g: (0, 0)),
        ],
        out_specs=pl.BlockSpec((NP, XB), lambda g: (0, g)),
        out_shape=jax.ShapeDtypeStruct((NP, B), jnp.int32),
    )(X, jnp.asarray(m))
    idx_flat = idx_t.reshape(NP * B)

    mesh = plsc.VectorSubcoreMesh(core_axis_name="c", subcore_axis_name="s")
    scratch = (
        [pltpu.VMEM((BLK,), jnp.int32) for _ in range(NRING)]
        + [pltpu.VMEM((BLK, 2 * D), jnp.float32) for _ in range(NRING)]
        + [pltpu.VMEM_SHARED((NROWP, 2 * D), jnp.float32)]
        + [pltpu.SemaphoreType.DMA for _ in range(3 * NRING)]
    )

    nblk_tr = CBATCH // CB
    buf = None
    for c in range(NCHUNK):
        gather = pl.kernel(
            _make_gather_body(c),
            mesh=mesh,
            out_type=jax.ShapeDtypeStruct((CBP, 2 * D), jnp.float32),
            scratch_types=scratch,
            compiler_params=pltpu.CompilerParams(needs_layout_passes=False),
        )
        pc = gather(idx_flat, tp).reshape(NP, CBATCH, 2 * D)
        if buf is None:
            buf = pl.pallas_call(
                _tr_body,
                grid=(nblk_tr,),
                in_specs=[pl.BlockSpec((NP, CB, 2 * D), lambda g: (0, g, 0))],
                out_specs=pl.BlockSpec((NP, 2 * D, CB), lambda g: (0, 0, g)),
                out_shape=jax.ShapeDtypeStruct((NP, 2 * D, B), jnp.float32),
            )(pc)
        else:
            buf = pl.pallas_call(
                _tr_body_alias,
                grid=(nblk_tr,),
                in_specs=[
                    pl.BlockSpec(memory_space=pltpu.MemorySpace.HBM),
                    pl.BlockSpec((NP, CB, 2 * D), lambda g: (0, g, 0)),
                ],
                out_specs=pl.BlockSpec(
                    (NP, 2 * D, CB),
                    lambda g, cc=c: (0, 0, cc * nblk_tr + g)),
                out_shape=jax.ShapeDtypeStruct((NP, 2 * D, B), jnp.float32),
                input_output_aliases={0: 0},
            )(buf, pc)

    return jnp.transpose(buf.reshape(F, D, B), (2, 0, 1))
